# baseline (speedup 1.0000x reference)
import functools

import jax
import jax.numpy as jnp
from jax import lax
from jax.experimental import pallas as pl
from jax.experimental.pallas import tpu as pltpu
from jax.experimental.pallas import tpu_sc as plsc


def _sc_gather(table, idx):
    B = idx.shape[0]
    _, D = table.shape
    info = plsc.get_sparse_core_info()
    nw = info.num_cores * info.num_subcores
    b_per_w = B // nw
    mesh = plsc.VectorSubcoreMesh(core_axis_name="c", subcore_axis_name="s")

    @functools.partial(
        pl.kernel,
        mesh=mesh,
        out_type=jax.ShapeDtypeStruct((B, D), jnp.float32),
        scratch_types=[
            pltpu.VMEM((b_per_w,), jnp.int32),
            pltpu.VMEM((b_per_w, D), jnp.float32),
            pltpu.SemaphoreType.DMA,
        ],
        compiler_params=pltpu.CompilerParams(use_tc_tiling_on_sc=False),
    )
    def gather_kernel(table_hbm, idx_hbm, out_hbm, idx_v, rows_v, sem):
        wid = lax.axis_index("s") * info.num_cores + lax.axis_index("c")
        base = wid * b_per_w
        pltpu.sync_copy(idx_hbm.at[pl.ds(base, b_per_w)], idx_v)
        pltpu.async_copy(table_hbm.at[idx_v], rows_v, sem).wait()
        pltpu.sync_copy(rows_v, out_hbm.at[pl.ds(base, b_per_w)])

    return gather_kernel(table, idx)


_TILE = 2048


def _matmul_tile(emb_ref, w_ref):
    return lax.dot_general(
        emb_ref[:], w_ref[:], (((1,), (1,)), ((), ())),
        preferred_element_type=jnp.float32)


def _stats_body(nt, V, emb_ref, w_ref, norm_ref, m_ref, s_ref):
    t = pl.program_id(0)

    @pl.when(t == 0)
    def _init():
        m_ref[:] = jnp.full_like(m_ref, -jnp.inf)
        s_ref[:] = jnp.zeros_like(s_ref)

    logits = _matmul_tile(emb_ref, w_ref)

    def _update(lm):
        tile_max = jnp.max(lm, axis=1, keepdims=True)
        m_old = m_ref[:]
        m_new = jnp.maximum(m_old, tile_max)
        s_ref[:] = s_ref[:] * jnp.exp(m_old - m_new) + jnp.sum(
            jnp.exp(lm - m_new), axis=1, keepdims=True)
        m_ref[:] = m_new

    @pl.when(t < nt - 1)
    def _full_tile():
        _update(logits)

    @pl.when(t == nt - 1)
    def _tail_tile():
        col = t * _TILE + lax.broadcasted_iota(jnp.int32, logits.shape, 1)
        _update(jnp.where(col < V, logits, -jnp.inf))
        norm_ref[:] = m_ref[:] + jnp.log(s_ref[:])


_NBUF = 8
_TILE_O = 1024


_ROWS_O = 32


def _out_body(emb_ref, w_ref, norm_ref, out_ref):
    out_ref[:] = lax.dot_general(
        emb_ref[:], w_ref[:], (((1,), (1,)), ((), ())),
        preferred_element_type=jnp.float32) - norm_ref[:]


def kernel(inputs, emb_table, out_weight):
    V, D = out_weight.shape
    B = inputs.shape[0]
    nt = pl.cdiv(V, _TILE)

    embeds = _sc_gather(emb_table, inputs)
    emb16 = embeds.astype(jnp.bfloat16)
    w16 = out_weight.astype(jnp.bfloat16)

    norm = pl.pallas_call(
        functools.partial(_stats_body, nt, V),
        grid=(nt,),
        in_specs=[
            pl.BlockSpec((B, D), lambda t: (0, 0)),
            pl.BlockSpec((_TILE, D), lambda t: (t, 0)),
        ],
        out_specs=pl.BlockSpec((B, 1), lambda t: (0, 0)),
        out_shape=jax.ShapeDtypeStruct((B, 1), jnp.float32),
        scratch_shapes=[
            pltpu.VMEM((B, 1), jnp.float32),
            pltpu.VMEM((B, 1), jnp.float32),
        ],
    )(emb16, w16)

    log_probs = pl.pallas_call(
        _out_body,
        grid=(B // _ROWS_O,),
        in_specs=[
            pl.BlockSpec((_ROWS_O, D), lambda i: (i, 0)),
            pl.BlockSpec((V, D), lambda i: (0, 0)),
            pl.BlockSpec((_ROWS_O, 1), lambda i: (i, 0)),
        ],
        out_specs=pl.BlockSpec((_ROWS_O, V), lambda i: (i, 0)),
        out_shape=jax.ShapeDtypeStruct((B, V), jnp.float32),
    )(emb16, w16, norm)

    return log_probs


# speedup vs baseline: 1.1873x; 1.1873x over previous
import functools

import jax
import jax.numpy as jnp
from jax import lax
from jax.experimental import pallas as pl
from jax.experimental.pallas import tpu as pltpu
from jax.experimental.pallas import tpu_sc as plsc


def _sc_gather(table, idx):
    B = idx.shape[0]
    _, D = table.shape
    info = plsc.get_sparse_core_info()
    nw = info.num_cores * info.num_subcores
    b_per_w = B // nw
    mesh = plsc.VectorSubcoreMesh(core_axis_name="c", subcore_axis_name="s")

    @functools.partial(
        pl.kernel,
        mesh=mesh,
        out_type=jax.ShapeDtypeStruct((B, D), jnp.float32),
        scratch_types=[
            pltpu.VMEM((b_per_w,), jnp.int32),
            pltpu.VMEM((b_per_w, D), jnp.float32),
            pltpu.SemaphoreType.DMA,
        ],
        compiler_params=pltpu.CompilerParams(use_tc_tiling_on_sc=False),
    )
    def gather_kernel(table_hbm, idx_hbm, out_hbm, idx_v, rows_v, sem):
        wid = lax.axis_index("s") * info.num_cores + lax.axis_index("c")
        base = wid * b_per_w
        pltpu.sync_copy(idx_hbm.at[pl.ds(base, b_per_w)], idx_v)
        pltpu.async_copy(table_hbm.at[idx_v], rows_v, sem).wait()
        pltpu.sync_copy(rows_v, out_hbm.at[pl.ds(base, b_per_w)])

    return gather_kernel(table, idx)


_TILE = 2048


def _matmul_tile(emb_ref, w_ref):
    return lax.dot_general(
        emb_ref[:], w_ref[:], (((1,), (1,)), ((), ())),
        preferred_element_type=jnp.float32)


def _stats_body(nt, V, emb_ref, w_ref, norm_ref, m_ref, s_ref):
    t = pl.program_id(0)

    @pl.when(t == 0)
    def _init():
        m_ref[:] = jnp.full_like(m_ref, -jnp.inf)
        s_ref[:] = jnp.zeros_like(s_ref)

    logits = _matmul_tile(emb_ref, w_ref)

    def _update(lm):
        tile_max = jnp.max(lm, axis=1, keepdims=True)
        m_old = m_ref[:]
        m_new = jnp.maximum(m_old, tile_max)
        s_ref[:] = s_ref[:] * jnp.exp(m_old - m_new) + jnp.sum(
            jnp.exp(lm - m_new), axis=1, keepdims=True)
        m_ref[:] = m_new

    @pl.when(t < nt - 1)
    def _full_tile():
        _update(logits)

    @pl.when(t == nt - 1)
    def _tail_tile():
        col = t * _TILE + lax.broadcasted_iota(jnp.int32, logits.shape, 1)
        _update(jnp.where(col < V, logits, -jnp.inf))
        norm_ref[:] = m_ref[:] + jnp.log(s_ref[:])


_NBUF = 8
_TILE_O = 1024


_NSPLIT = 4


def _out_ring_body(nt, B, emb_ref, w_ref, norm_ref, out_hbm, ring, sems):
    t = pl.program_id(0)
    slot = lax.rem(t, _NBUF)
    rows = B // _NSPLIT

    def _copies(step, s):
        return [
            pltpu.make_async_copy(
                ring.at[s, pl.ds(r * rows, rows), :],
                out_hbm.at[pl.ds(r * rows, rows),
                           pl.ds(step * _TILE_O, _TILE_O)],
                sems.at[s, r])
            for r in range(_NSPLIT)
        ]

    @pl.when(t >= _NBUF)
    def _retire():
        for c in _copies(t - _NBUF, slot):
            c.wait()

    ring[slot] = lax.dot_general(
        emb_ref[:], w_ref[:], (((1,), (1,)), ((), ())),
        preferred_element_type=jnp.float32) - norm_ref[:]

    for c in _copies(t, slot):
        c.start()

    @pl.when(t == nt - 1)
    def _drain():
        for k in range(_NBUF):
            tt = t - _NBUF + 1 + k
            s = lax.rem(tt, _NBUF)
            for c in _copies(tt, s):
                c.wait()


def _tail_body(emb_ref, w_ref, norm_ref, big_ref, out_ref):
    del big_ref
    out_ref[:] = lax.dot_general(
        emb_ref[:], w_ref[:], (((1,), (1,)), ((), ())),
        preferred_element_type=jnp.float32) - norm_ref[:]


def kernel(inputs, emb_table, out_weight):
    V, D = out_weight.shape
    B = inputs.shape[0]
    nt = pl.cdiv(V, _TILE)

    embeds = _sc_gather(emb_table, inputs)
    emb16 = embeds.astype(jnp.bfloat16)
    w16 = out_weight.astype(jnp.bfloat16)

    norm = pl.pallas_call(
        functools.partial(_stats_body, nt, V),
        grid=(nt,),
        in_specs=[
            pl.BlockSpec((B, D), lambda t: (0, 0)),
            pl.BlockSpec((_TILE, D), lambda t: (t, 0)),
        ],
        out_specs=pl.BlockSpec((B, 1), lambda t: (0, 0)),
        out_shape=jax.ShapeDtypeStruct((B, 1), jnp.float32),
        scratch_shapes=[
            pltpu.VMEM((B, 1), jnp.float32),
            pltpu.VMEM((B, 1), jnp.float32),
        ],
    )(emb16, w16)

    nt_full = V // _TILE_O
    main = pl.pallas_call(
        functools.partial(_out_ring_body, nt_full, B),
        grid=(nt_full,),
        in_specs=[
            pl.BlockSpec((B, D), lambda t: (0, 0)),
            pl.BlockSpec((_TILE_O, D), lambda t: (t, 0)),
            pl.BlockSpec((B, 1), lambda t: (0, 0)),
        ],
        out_specs=pl.BlockSpec(memory_space=pl.ANY),
        out_shape=jax.ShapeDtypeStruct((B, V), jnp.float32),
        scratch_shapes=[
            pltpu.VMEM((_NBUF, B, _TILE_O), jnp.float32),
            pltpu.SemaphoreType.DMA((_NBUF, _NSPLIT)),
        ],
    )(emb16, w16, norm)

    log_probs = pl.pallas_call(
        _tail_body,
        grid=(1,),
        in_specs=[
            pl.BlockSpec((B, D), lambda t: (0, 0)),
            pl.BlockSpec((_TILE_O, D), lambda t: (nt_full, 0)),
            pl.BlockSpec((B, 1), lambda t: (0, 0)),
            pl.BlockSpec(memory_space=pl.ANY),
        ],
        out_specs=pl.BlockSpec((B, _TILE_O), lambda t: (0, nt_full)),
        out_shape=jax.ShapeDtypeStruct((B, V), jnp.float32),
        input_output_aliases={3: 0},
    )(emb16, w16, norm, main)

    return log_probs


# speedup vs baseline: 1.1893x; 1.0017x over previous
import functools

import jax
import jax.numpy as jnp
from jax import lax
from jax.experimental import pallas as pl
from jax.experimental.pallas import tpu as pltpu
from jax.experimental.pallas import tpu_sc as plsc


def _sc_gather(table, idx):
    B = idx.shape[0]
    _, D = table.shape
    info = plsc.get_sparse_core_info()
    nw = info.num_cores * info.num_subcores
    b_per_w = B // nw
    mesh = plsc.VectorSubcoreMesh(core_axis_name="c", subcore_axis_name="s")

    @functools.partial(
        pl.kernel,
        mesh=mesh,
        out_type=jax.ShapeDtypeStruct((B, D), jnp.float32),
        scratch_types=[
            pltpu.VMEM((b_per_w,), jnp.int32),
            pltpu.VMEM((b_per_w, D), jnp.float32),
            pltpu.SemaphoreType.DMA,
        ],
        compiler_params=pltpu.CompilerParams(use_tc_tiling_on_sc=False),
    )
    def gather_kernel(table_hbm, idx_hbm, out_hbm, idx_v, rows_v, sem):
        wid = lax.axis_index("s") * info.num_cores + lax.axis_index("c")
        base = wid * b_per_w
        pltpu.sync_copy(idx_hbm.at[pl.ds(base, b_per_w)], idx_v)
        pltpu.async_copy(table_hbm.at[idx_v], rows_v, sem).wait()
        pltpu.sync_copy(rows_v, out_hbm.at[pl.ds(base, b_per_w)])

    return gather_kernel(table, idx)


_TILE = 2048


def _matmul_tile(emb_ref, w_ref):
    return lax.dot_general(
        emb_ref[:], w_ref[:], (((1,), (1,)), ((), ())),
        preferred_element_type=jnp.float32)


def _stats_body(nt, V, emb_ref, w_ref, norm_ref, m_ref, s_ref):
    t = pl.program_id(0)

    @pl.when(t == 0)
    def _init():
        m_ref[:] = jnp.full_like(m_ref, -jnp.inf)
        s_ref[:] = jnp.zeros_like(s_ref)

    logits = _matmul_tile(emb_ref, w_ref)

    def _update(lm):
        tile_max = jnp.max(lm, axis=1, keepdims=True)
        m_old = m_ref[:]
        m_new = jnp.maximum(m_old, tile_max)
        s_ref[:] = s_ref[:] * jnp.exp(m_old - m_new) + jnp.sum(
            jnp.exp(lm - m_new), axis=1, keepdims=True)
        m_ref[:] = m_new

    @pl.when(t < nt - 1)
    def _full_tile():
        _update(logits)

    @pl.when(t == nt - 1)
    def _tail_tile():
        col = t * _TILE + lax.broadcasted_iota(jnp.int32, logits.shape, 1)
        _update(jnp.where(col < V, logits, -jnp.inf))
        norm_ref[:] = m_ref[:] + jnp.log(s_ref[:])


_NBUF = 8
_TILE_O = 1024


_NSPLIT = 4


def _out_ring_body(nt, B, emb_ref, w_ref, norm_ref, out_hbm, ring, sems):
    t = pl.program_id(0)
    slot = lax.rem(t, _NBUF)
    rows = B // _NSPLIT

    def _copies(step, s):
        return [
            pltpu.make_async_copy(
                ring.at[s, pl.ds(r * rows, rows), :],
                out_hbm.at[pl.ds(r * rows, rows),
                           pl.ds(step * _TILE_O, _TILE_O)],
                sems.at[s, r])
            for r in range(_NSPLIT)
        ]

    @pl.when(t >= _NBUF)
    def _retire():
        for c in _copies(t - _NBUF, slot):
            c.wait()

    ring[slot] = lax.dot_general(
        emb_ref[:], w_ref[:], (((1,), (1,)), ((), ())),
        preferred_element_type=jnp.float32) - norm_ref[:]

    for r, c in enumerate(_copies(t, slot)):
        c.start(priority=r % 2)

    @pl.when(t == nt - 1)
    def _drain():
        for k in range(_NBUF):
            tt = t - _NBUF + 1 + k
            s = lax.rem(tt, _NBUF)
            for c in _copies(tt, s):
                c.wait()


def _tail_body(emb_ref, w_ref, norm_ref, big_ref, out_ref):
    del big_ref
    out_ref[:] = lax.dot_general(
        emb_ref[:], w_ref[:], (((1,), (1,)), ((), ())),
        preferred_element_type=jnp.float32) - norm_ref[:]


def kernel(inputs, emb_table, out_weight):
    V, D = out_weight.shape
    B = inputs.shape[0]
    nt = pl.cdiv(V, _TILE)

    embeds = _sc_gather(emb_table, inputs)
    emb16 = embeds.astype(jnp.bfloat16)
    w16 = out_weight.astype(jnp.bfloat16)

    norm = pl.pallas_call(
        functools.partial(_stats_body, nt, V),
        grid=(nt,),
        in_specs=[
            pl.BlockSpec((B, D), lambda t: (0, 0)),
            pl.BlockSpec((_TILE, D), lambda t: (t, 0)),
        ],
        out_specs=pl.BlockSpec((B, 1), lambda t: (0, 0)),
        out_shape=jax.ShapeDtypeStruct((B, 1), jnp.float32),
        scratch_shapes=[
            pltpu.VMEM((B, 1), jnp.float32),
            pltpu.VMEM((B, 1), jnp.float32),
        ],
    )(emb16, w16)

    nt_full = V // _TILE_O
    main = pl.pallas_call(
        functools.partial(_out_ring_body, nt_full, B),
        grid=(nt_full,),
        in_specs=[
            pl.BlockSpec((B, D), lambda t: (0, 0)),
            pl.BlockSpec((_TILE_O, D), lambda t: (t, 0)),
            pl.BlockSpec((B, 1), lambda t: (0, 0)),
        ],
        out_specs=pl.BlockSpec(memory_space=pl.ANY),
        out_shape=jax.ShapeDtypeStruct((B, V), jnp.float32),
        scratch_shapes=[
            pltpu.VMEM((_NBUF, B, _TILE_O), jnp.float32),
            pltpu.SemaphoreType.DMA((_NBUF, _NSPLIT)),
        ],
    )(emb16, w16, norm)

    log_probs = pl.pallas_call(
        _tail_body,
        grid=(1,),
        in_specs=[
            pl.BlockSpec((B, D), lambda t: (0, 0)),
            pl.BlockSpec((_TILE_O, D), lambda t: (nt_full, 0)),
            pl.BlockSpec((B, 1), lambda t: (0, 0)),
            pl.BlockSpec(memory_space=pl.ANY),
        ],
        out_specs=pl.BlockSpec((B, _TILE_O), lambda t: (0, nt_full)),
        out_shape=jax.ShapeDtypeStruct((B, V), jnp.float32),
        input_output_aliases={3: 0},
    )(emb16, w16, norm, main)

    return log_probs


# speedup vs baseline: 1.2205x; 1.0262x over previous
import functools

import jax
import jax.numpy as jnp
from jax import lax
from jax.experimental import pallas as pl
from jax.experimental.pallas import tpu as pltpu
from jax.experimental.pallas import tpu_sc as plsc


def _sc_gather(table, idx):
    B = idx.shape[0]
    _, D = table.shape
    info = plsc.get_sparse_core_info()
    nw = info.num_cores * info.num_subcores
    b_per_w = B // nw
    mesh = plsc.VectorSubcoreMesh(core_axis_name="c", subcore_axis_name="s")

    @functools.partial(
        pl.kernel,
        mesh=mesh,
        out_type=jax.ShapeDtypeStruct((B, D), jnp.float32),
        scratch_types=[
            pltpu.VMEM((b_per_w,), jnp.int32),
            pltpu.VMEM((b_per_w, D), jnp.float32),
            pltpu.SemaphoreType.DMA,
        ],
        compiler_params=pltpu.CompilerParams(use_tc_tiling_on_sc=False),
    )
    def gather_kernel(table_hbm, idx_hbm, out_hbm, idx_v, rows_v, sem):
        wid = lax.axis_index("s") * info.num_cores + lax.axis_index("c")
        base = wid * b_per_w
        pltpu.sync_copy(idx_hbm.at[pl.ds(base, b_per_w)], idx_v)
        pltpu.async_copy(table_hbm.at[idx_v], rows_v, sem).wait()
        pltpu.sync_copy(rows_v, out_hbm.at[pl.ds(base, b_per_w)])

    return gather_kernel(table, idx)


_TILE = 2048


def _matmul_tile(emb_ref, w_ref):
    return lax.dot_general(
        emb_ref[:], w_ref[:], (((1,), (1,)), ((), ())),
        preferred_element_type=jnp.float32)


def _stats_body(nt, V, emb_ref, w_ref, norm_ref, m_ref, s_ref):
    t = pl.program_id(0)

    @pl.when(t == 0)
    def _init():
        m_ref[:] = jnp.full_like(m_ref, -jnp.inf)
        s_ref[:] = jnp.zeros_like(s_ref)

    logits = _matmul_tile(emb_ref, w_ref)

    def _update(lm):
        tile_max = jnp.max(lm, axis=1, keepdims=True)
        m_old = m_ref[:]
        m_new = jnp.maximum(m_old, tile_max)
        s_ref[:] = s_ref[:] * jnp.exp(m_old - m_new) + jnp.sum(
            jnp.exp(lm - m_new), axis=1, keepdims=True)
        m_ref[:] = m_new

    @pl.when(t < nt - 1)
    def _full_tile():
        _update(logits)

    @pl.when(t == nt - 1)
    def _tail_tile():
        col = t * _TILE + lax.broadcasted_iota(jnp.int32, logits.shape, 1)
        _update(jnp.where(col < V, logits, -jnp.inf))
        norm_ref[:] = m_ref[:] + jnp.log(s_ref[:])


_NBUF = 8
_TILE_O = 512


def _out_ring_body(nsteps, emb_ref, w_ref, norm_ref, out_hbm, *rest):
    bufs, sems = rest[:_NBUF], rest[_NBUF]
    t = pl.program_id(0)

    def _copy(step, b):
        return pltpu.make_async_copy(
            bufs[b],
            out_hbm.at[:, pl.ds((step * _NBUF + b) * _TILE_O, _TILE_O)],
            sems.at[b])

    for b in range(_NBUF):
        @pl.when(t > 0)
        def _retire(b=b):
            _copy(t - 1, b).wait()

        bufs[b][:] = lax.dot_general(
            emb_ref[:], w_ref[pl.ds(b * _TILE_O, _TILE_O), :],
            (((1,), (1,)), ((), ())),
            preferred_element_type=jnp.float32) - norm_ref[:]
        _copy(t, b).start()

    @pl.when(t == nsteps - 1)
    def _drain():
        for b in range(_NBUF):
            _copy(t, b).wait()


def _tail_body(emb_ref, w_ref, norm_ref, big_ref, out_ref):
    del big_ref
    out_ref[:] = lax.dot_general(
        emb_ref[:], w_ref[:], (((1,), (1,)), ((), ())),
        preferred_element_type=jnp.float32) - norm_ref[:]


def kernel(inputs, emb_table, out_weight):
    V, D = out_weight.shape
    B = inputs.shape[0]
    nt = pl.cdiv(V, _TILE)

    embeds = _sc_gather(emb_table, inputs)
    emb16 = embeds.astype(jnp.bfloat16)
    w16 = out_weight.astype(jnp.bfloat16)

    norm = pl.pallas_call(
        functools.partial(_stats_body, nt, V),
        grid=(nt,),
        in_specs=[
            pl.BlockSpec((B, D), lambda t: (0, 0)),
            pl.BlockSpec((_TILE, D), lambda t: (t, 0)),
        ],
        out_specs=pl.BlockSpec((B, 1), lambda t: (0, 0)),
        out_shape=jax.ShapeDtypeStruct((B, 1), jnp.float32),
        scratch_shapes=[
            pltpu.VMEM((B, 1), jnp.float32),
            pltpu.VMEM((B, 1), jnp.float32),
        ],
    )(emb16, w16)

    chunk = _NBUF * _TILE_O
    nsteps = V // chunk
    tail_cols = V - nsteps * chunk
    main = pl.pallas_call(
        functools.partial(_out_ring_body, nsteps),
        grid=(nsteps,),
        in_specs=[
            pl.BlockSpec((B, D), lambda t: (0, 0)),
            pl.BlockSpec((chunk, D), lambda t: (t, 0)),
            pl.BlockSpec((B, 1), lambda t: (0, 0)),
        ],
        out_specs=pl.BlockSpec(memory_space=pl.ANY),
        out_shape=jax.ShapeDtypeStruct((B, V), jnp.float32),
        scratch_shapes=(
            [pltpu.VMEM((B, _TILE_O), jnp.float32) for _ in range(_NBUF)]
            + [pltpu.SemaphoreType.DMA((_NBUF,))]),
    )(emb16, w16, norm)

    tail_tile = chunk // 2
    log_probs = pl.pallas_call(
        _tail_body,
        grid=(1,),
        in_specs=[
            pl.BlockSpec((B, D), lambda t: (0, 0)),
            pl.BlockSpec((tail_tile, D), lambda t: (nsteps * chunk // tail_tile, 0)),
            pl.BlockSpec((B, 1), lambda t: (0, 0)),
            pl.BlockSpec(memory_space=pl.ANY),
        ],
        out_specs=pl.BlockSpec((B, tail_tile),
                               lambda t: (0, nsteps * chunk // tail_tile)),
        out_shape=jax.ShapeDtypeStruct((B, V), jnp.float32),
        input_output_aliases={3: 0},
    )(emb16, w16, norm, main)

    return log_probs
